# Initial kernel scaffold; baseline (speedup 1.0000x reference)
import jax
import jax.numpy as jnp
from jax.experimental import pallas as pl


def kernel(adj_indices, adj_values, sub1_indices, sub1_values, sub2_indices, sub2_values, users, items, neg_items, user_emb, item_emb, du_W1, du_b1, du_W2, du_b2, di_W1, di_b1, di_W2, di_b2):
    raise NotImplementedError("write your pallas kernel here")


# speedup vs baseline: 2.4973x; 2.4973x over previous
import jax
import jax.numpy as jnp
from jax import lax
from jax.experimental import pallas as pl
from jax.experimental.pallas import tpu as pltpu
from jax.experimental.pallas import tpu_sc as plsc

N_USERS = 25000
N_ITEMS = 25000
NN = 50000
D = 64
E = 800000
B = 1024

HALF = 25000
ACC_ROWS = 25008
CHUNK = 128
NCHUNKS = E // CHUNK
FULL = NCHUNKS // 16
EXTRA = NCHUNKS - FULL * 16
STRIPE = ACC_ROWS // 16
OUT_STRIPE = 1568
NBUF = 3

_MESH = plsc.VectorSubcoreMesh(core_axis_name="c", subcore_axis_name="s")


def _spmm_body(dst_hbm, src_hbm, val_hbm, x_hbm, out_hbm, acc_sh,
               src0, src1, src2, dl0, dl1, dl2, val0, val1, val2,
               rows0, rows1, rows2,
               sin0, sin1, sin2, srow0, srow1, srow2, ssc0, ssc1, ssc2):
    c = lax.axis_index("c")
    s = lax.axis_index("s")
    base_row = c * HALF

    srcb = (src0, src1, src2)
    dlb = (dl0, dl1, dl2)
    valb = (val0, val1, val2)
    rowsb = (rows0, rows1, rows2)
    sin = (sin0, sin1, sin2)
    srow = (srow0, srow1, srow2)
    ssc = (ssc0, ssc1, ssc2)

    @pl.loop(0, CHUNK)
    def _(e):
        for k in range(4):
            rows0[e, pl.ds(k * 16, 16)] = jnp.zeros((16,), jnp.float32)

    z0 = s * STRIPE
    nz_full = STRIPE // CHUNK
    nz_rem = STRIPE - nz_full * CHUNK

    @pl.loop(0, nz_full)
    def _(i):
        pltpu.sync_copy(rows0, acc_sh.at[pl.ds(z0 + i * CHUNK, CHUNK)])

    if nz_rem:
        pltpu.sync_copy(rows0.at[pl.ds(0, nz_rem)],
                        acc_sh.at[pl.ds(z0 + nz_full * CHUNK, nz_rem)])

    plsc.subcore_barrier()

    def fetch(m, b):
        e0 = m * CHUNK
        pltpu.async_copy(src_hbm.at[pl.ds(e0, CHUNK)], srcb[b], sin[b])
        pltpu.async_copy(dst_hbm.at[pl.ds(e0, CHUNK)], dlb[b], sin[b])
        pltpu.async_copy(val_hbm.at[pl.ds(e0, CHUNK)], valb[b], sin[b])

    def fetch_wait(b):
        pltpu.make_async_copy(src_hbm.at[pl.ds(0, CHUNK)], srcb[b], sin[b]).wait()
        pltpu.make_async_copy(dst_hbm.at[pl.ds(0, CHUNK)], dlb[b], sin[b]).wait()
        pltpu.make_async_copy(val_hbm.at[pl.ds(0, CHUNK)], valb[b], sin[b]).wait()

    def gather_start(b):
        pltpu.async_copy(x_hbm.at[srcb[b]], rowsb[b], srow[b])

    def gather_wait(b):
        pltpu.make_async_copy(x_hbm.at[srcb[b]], rowsb[b], srow[b]).wait()

    def scat_start(b):
        pltpu.async_copy(rowsb[b], acc_sh.at[dlb[b]], ssc[b], add=True)

    def scat_wait(b):
        pltpu.make_async_copy(rowsb[b], acc_sh.at[dlb[b]], ssc[b]).wait()

    def compute(b):
        @pl.loop(0, CHUNK, step=16)
        def _(i):
            dd = dlb[b][pl.ds(i, 16)]
            loc = dd - base_row
            oob = (loc < 0) | (loc >= HALF)
            dlb[b][pl.ds(i, 16)] = jnp.where(oob, HALF, loc)

        @pl.loop(0, CHUNK, step=16)
        def _(i):
            v16 = valb[b][pl.ds(i, 16)]
            for u in range(16):
                v = v16[u]
                for k in range(4):
                    sl = pl.ds(k * 16, 16)
                    rowsb[b][i + u, sl] = rowsb[b][i + u, sl] * v

    def mid(g):
        return g * 16 + s

    fetch(mid(0), 0)
    fetch_wait(0)
    gather_start(0)
    fetch(mid(1), 1)

    @pl.loop(0, FULL, step=NBUF)
    def _(g0):
        for b in range(NBUF):
            g = g0 + b
            bp1 = (b + 1) % NBUF
            bp2 = (b + 2) % NBUF

            @pl.when(g + 1 < FULL)
            def _():
                fetch_wait(bp1)

                @pl.when(g >= 2)
                def _():
                    scat_wait(bp1)

                gather_start(bp1)

            gather_wait(b)

            @pl.when(g + 2 < FULL)
            def _():
                fetch(mid(g + 2), bp2)

            compute(b)
            scat_start(b)

    scat_wait(0)
    scat_wait(1)
    scat_wait(2)

    @pl.when(s < EXTRA)
    def _():
        fetch(mid(FULL), 0)
        fetch_wait(0)
        gather_start(0)
        gather_wait(0)
        compute(0)
        scat_start(0)
        scat_wait(0)

    plsc.subcore_barrier()

    @pl.when(s < 15)
    def _():
        pltpu.sync_copy(
            acc_sh.at[pl.ds(s * OUT_STRIPE, OUT_STRIPE)],
            out_hbm.at[pl.ds(base_row + s * OUT_STRIPE, OUT_STRIPE)])

    @pl.when(s == 15)
    def _():
        last = HALF - 15 * OUT_STRIPE
        pltpu.sync_copy(
            acc_sh.at[pl.ds(15 * OUT_STRIPE, last)],
            out_hbm.at[pl.ds(base_row + 15 * OUT_STRIPE, last)])


def _spmm(dst, src, values, x):
    f = pl.kernel(
        _spmm_body,
        out_type=jax.ShapeDtypeStruct((NN, D), jnp.float32),
        mesh=_MESH,
        compiler_params=pltpu.CompilerParams(use_tc_tiling_on_sc=False),
        scratch_types=(
            [pltpu.VMEM_SHARED((ACC_ROWS, D), jnp.float32)]
            + [pltpu.VMEM((CHUNK,), jnp.int32)] * 6
            + [pltpu.VMEM((CHUNK,), jnp.float32)] * 3
            + [pltpu.VMEM((CHUNK, D), jnp.float32)] * 3
            + [pltpu.SemaphoreType.DMA] * 9
        ),
    )
    return f(dst, src, values, x)


GB = 128


def _head_body(ue0, ie0, u1, u2, i1, i2, users, items, negs,
               o_u1s, o_u2s, o_i1s, o_i2s, o_ue, o_ie, o_ne,
               uidx, iidx, nidx, buf):
    c = lax.axis_index("c")
    s = lax.axis_index("s")
    w = s * 2 + c

    @pl.when(w < B // GB)
    def _():
        b0 = w * GB
        pltpu.sync_copy(users.at[pl.ds(b0, GB)], uidx)
        pltpu.sync_copy(items.at[pl.ds(b0, GB)], iidx)
        pltpu.sync_copy(negs.at[pl.ds(b0, GB)], nidx)
        for tbl, idx, out in ((u1, uidx, o_u1s), (u2, uidx, o_u2s),
                              (i1, iidx, o_i1s), (i2, iidx, o_i2s),
                              (ue0, uidx, o_ue), (ie0, iidx, o_ie),
                              (ie0, nidx, o_ne)):
            pltpu.sync_copy(tbl.at[idx], buf)
            pltpu.sync_copy(buf, out.at[pl.ds(b0, GB)])


def _head(ue0, ie0, u1, u2, i1, i2, users, items, negs):
    rows = jax.ShapeDtypeStruct((B, D), jnp.float32)
    f = pl.kernel(
        _head_body,
        out_type=(rows,) * 7,
        mesh=_MESH,
        compiler_params=pltpu.CompilerParams(use_tc_tiling_on_sc=False),
        scratch_types=[
            pltpu.VMEM((GB,), jnp.int32),
            pltpu.VMEM((GB,), jnp.int32),
            pltpu.VMEM((GB,), jnp.int32),
            pltpu.VMEM((GB, D), jnp.float32),
        ],
    )
    return f(ue0, ie0, u1, u2, i1, i2, users, items, negs)


def _avg4_body(a, b, c, d, o):
    o[...] = (a[...] + b[...] + c[...] + d[...]) * 0.25


def _avg4n_body(a, b, c, d, o):
    m = (a[...] + b[...] + c[...] + d[...]) * 0.25
    n = jnp.sqrt(jnp.sum(m * m, axis=1, keepdims=True))
    o[...] = m / jnp.maximum(n, 1e-12)


def _avg4(x0, x1, x2, x3, norm):
    body = _avg4n_body if norm else _avg4_body
    blk = pl.BlockSpec((1000, D), lambda i: (i, 0))
    return pl.pallas_call(
        body,
        grid=(NN // 1000,),
        in_specs=[blk] * 4,
        out_specs=blk,
        out_shape=jax.ShapeDtypeStruct((NN, D), jnp.float32),
    )(x0, x1, x2, x3)


def _denoise_body(x, nz, w1, b1, w2, b2, o):
    h = lax.dot_general(x[...] + nz[...], w1[...], (((1,), (0,)), ((), ())),
                        preferred_element_type=jnp.float32) + b1[...]
    h = jnp.maximum(h, 0.0)
    o[...] = lax.dot_general(h, w2[...], (((1,), (0,)), ((), ())),
                             preferred_element_type=jnp.float32) + b2[...]


def _denoise(x, nz, w1, b1, w2, b2):
    R = 1000
    return pl.pallas_call(
        _denoise_body,
        grid=(N_USERS // R,),
        in_specs=[pl.BlockSpec((R, D), lambda i: (i, 0)),
                  pl.BlockSpec((R, D), lambda i: (i, 0)),
                  pl.BlockSpec((D, 2 * D), lambda i: (0, 0)),
                  pl.BlockSpec((1, 2 * D), lambda i: (0, 0)),
                  pl.BlockSpec((2 * D, D), lambda i: (0, 0)),
                  pl.BlockSpec((1, D), lambda i: (0, 0))],
        out_specs=pl.BlockSpec((R, D), lambda i: (i, 0)),
        out_shape=jax.ShapeDtypeStruct((N_USERS, D), jnp.float32),
    )(x, nz, w1, b1.reshape(1, -1), w2, b2.reshape(1, -1))


def _sup_body(ue, ie, ne, o):
    o[...] = jnp.sum(ue[...] * (ie[...] - ne[...]), axis=1, keepdims=True)


def _sup(ue, ie, ne):
    return pl.pallas_call(
        _sup_body,
        out_shape=jax.ShapeDtypeStruct((B, 1), jnp.float32),
    )(ue, ie, ne).reshape(B)


def _ssl_body(u1s, u2s, tbl, o):
    pos = jnp.sum(u1s[...] * u2s[...], axis=1, keepdims=True)
    o[...] = lax.dot_general(u1s[...], tbl[...], (((1,), (1,)), ((), ())),
                             preferred_element_type=jnp.float32) - pos


def _ssl(u1s, u2s, tbl):
    TB = 1024
    NP = tbl.shape[0]
    return pl.pallas_call(
        _ssl_body,
        grid=(NP // TB,),
        in_specs=[pl.BlockSpec((B, D), lambda i: (0, 0)),
                  pl.BlockSpec((B, D), lambda i: (0, 0)),
                  pl.BlockSpec((TB, D), lambda i: (i, 0))],
        out_specs=pl.BlockSpec((B, TB), lambda i: (0, i)),
        out_shape=jax.ShapeDtypeStruct((B, NP), jnp.float32),
    )(u1s, u2s, tbl)


def kernel(adj_indices, adj_values, sub1_indices, sub1_values,
           sub2_indices, sub2_values, users, items, neg_items,
           user_emb, item_emb,
           du_W1, du_b1, du_W2, du_b2, di_W1, di_b1, di_W2, di_b2):
    adj_indices = adj_indices.astype(jnp.int32)
    sub1_indices = sub1_indices.astype(jnp.int32)
    sub2_indices = sub2_indices.astype(jnp.int32)
    users = users.astype(jnp.int32)
    items = items.astype(jnp.int32)
    negs = neg_items.astype(jnp.int32)

    x0 = jnp.concatenate([user_emb, item_emb], axis=0)

    def chain(idx, val, norm):
        dst, src = idx[0], idx[1]
        x1 = _spmm(dst, src, val, x0)
        x2 = _spmm(dst, src, val, x1)
        x3 = _spmm(dst, src, val, x2)
        return _avg4(x0, x1, x2, x3, norm)

    avg_adj = chain(adj_indices, adj_values, False)
    avg1 = chain(sub1_indices, sub1_values, True)
    avg2 = chain(sub2_indices, sub2_values, True)

    nkey = jax.random.key(42)
    nz_u = jax.random.normal(jax.random.fold_in(nkey, 0),
                             (N_USERS, D), dtype=jnp.float32) * 0.1
    nz_i = jax.random.normal(jax.random.fold_in(nkey, 1),
                             (N_ITEMS, D), dtype=jnp.float32) * 0.1

    ue0 = _denoise(avg_adj[:N_USERS], nz_u, du_W1, du_b1, du_W2, du_b2)
    ie0 = _denoise(avg_adj[N_USERS:], nz_i, di_W1, di_b1, di_W2, di_b2)

    u1, i1 = avg1[:N_USERS], avg1[N_USERS:]
    u2, i2 = avg2[:N_USERS], avg2[N_USERS:]

    u1s, u2s, i1s, i2s, u_e, i_e, n_e = _head(
        ue0, ie0, u1, u2, i1, i2, users, items, negs)

    sup = _sup(u_e, i_e, n_e)
    u2p = jnp.pad(u2, ((0, 600), (0, 0)))
    i2p = jnp.pad(i2, ((0, 600), (0, 0)))
    ssl_u = _ssl(u1s, u2s, u2p)[:, :N_USERS]
    ssl_i = _ssl(i1s, i2s, i2p)[:, :N_ITEMS]
    return sup, ssl_u, ssl_i


# speedup vs baseline: 5.8955x; 2.3607x over previous
import jax
import jax.numpy as jnp
from jax import lax
from jax.experimental import pallas as pl
from jax.experimental.pallas import tpu as pltpu
from jax.experimental.pallas import tpu_sc as plsc

N_USERS = 25000
N_ITEMS = 25000
NN = 50000
D = 64
E = 800000
B = 1024

DH = D // 2
CHUNK = 128
NCHUNKS = E // CHUNK
FULL = NCHUNKS // 16
EXTRA = NCHUNKS - FULL * 16
STRIPE = NN // 16
NBUF = 3

_MESH = plsc.VectorSubcoreMesh(core_axis_name="c", subcore_axis_name="s")


def _spmm_body(dst_hbm, src2_hbm, val_hbm, x_hbm, out_hbm, acc_sh,
               src0, src1, src2, dl0, dl1, dl2, val0, val1, val2,
               rows0, rows1, rows2,
               sin0, sin1, sin2, srow0, srow1, srow2, ssc0, ssc1, ssc2):
    c = lax.axis_index("c")
    s = lax.axis_index("s")

    srcb = (src0, src1, src2)
    dlb = (dl0, dl1, dl2)
    valb = (val0, val1, val2)
    rowsb = (rows0, rows1, rows2)
    sin = (sin0, sin1, sin2)
    srow = (srow0, srow1, srow2)
    ssc = (ssc0, ssc1, ssc2)

    @pl.loop(0, CHUNK)
    def _(e):
        for k in range(DH // 16):
            rows0[e, pl.ds(k * 16, 16)] = jnp.zeros((16,), jnp.float32)

    z0 = s * STRIPE
    nz_full = STRIPE // CHUNK
    nz_rem = STRIPE - nz_full * CHUNK

    for i in range(nz_full):
        pltpu.async_copy(rows0, acc_sh.at[pl.ds(z0 + i * CHUNK, CHUNK)], sin0)
    if nz_rem:
        pltpu.async_copy(rows0.at[pl.ds(0, nz_rem)],
                         acc_sh.at[pl.ds(z0 + nz_full * CHUNK, nz_rem)], sin0)
    for i in range(nz_full):
        pltpu.make_async_copy(
            rows0, acc_sh.at[pl.ds(z0 + i * CHUNK, CHUNK)], sin0).wait()
    if nz_rem:
        pltpu.make_async_copy(
            rows0.at[pl.ds(0, nz_rem)],
            acc_sh.at[pl.ds(z0 + nz_full * CHUNK, nz_rem)], sin0).wait()

    plsc.subcore_barrier()

    def fetch(m, b):
        e0 = m * CHUNK
        pltpu.async_copy(src2_hbm.at[c, pl.ds(e0, CHUNK)], srcb[b], sin[b])
        pltpu.async_copy(dst_hbm.at[pl.ds(e0, CHUNK)], dlb[b], sin[b])
        pltpu.async_copy(val_hbm.at[pl.ds(e0, CHUNK)], valb[b], sin[b])

    def fetch_wait(b):
        pltpu.make_async_copy(src2_hbm.at[0, pl.ds(0, CHUNK)], srcb[b], sin[b]).wait()
        pltpu.make_async_copy(dst_hbm.at[pl.ds(0, CHUNK)], dlb[b], sin[b]).wait()
        pltpu.make_async_copy(val_hbm.at[pl.ds(0, CHUNK)], valb[b], sin[b]).wait()

    def gather_start(b):
        pltpu.async_copy(x_hbm.at[srcb[b]], rowsb[b], srow[b])

    def gather_wait(b):
        pltpu.make_async_copy(x_hbm.at[srcb[b]], rowsb[b], srow[b]).wait()

    def scat_start(b):
        pltpu.async_copy(rowsb[b], acc_sh.at[dlb[b]], ssc[b], add=True)

    def scat_wait(b):
        pltpu.make_async_copy(rowsb[b], acc_sh.at[dlb[b]], ssc[b]).wait()

    def compute(b):
        @pl.loop(0, CHUNK, step=16)
        def _(i):
            v16 = valb[b][pl.ds(i, 16)]
            for u in range(16):
                v = v16[u]
                for k in range(DH // 16):
                    sl = pl.ds(k * 16, 16)
                    rowsb[b][i + u, sl] = rowsb[b][i + u, sl] * v

    def mid(g):
        return g * 16 + s

    fetch(mid(0), 0)
    fetch_wait(0)
    gather_start(0)
    fetch(mid(1), 1)

    @pl.loop(0, FULL, step=NBUF)
    def _(g0):
        for b in range(NBUF):
            g = g0 + b
            bp1 = (b + 1) % NBUF
            bp2 = (b + 2) % NBUF

            @pl.when(g + 1 < FULL)
            def _():
                fetch_wait(bp1)

                @pl.when(g >= 2)
                def _():
                    scat_wait(bp1)

                gather_start(bp1)

            gather_wait(b)

            @pl.when(g + 2 < FULL)
            def _():
                fetch(mid(g + 2), bp2)

            compute(b)
            scat_start(b)

    scat_wait(0)
    scat_wait(1)
    scat_wait(2)

    @pl.when(s < EXTRA)
    def _():
        fetch(mid(FULL), 0)
        fetch_wait(0)
        gather_start(0)
        gather_wait(0)
        compute(0)
        scat_start(0)
        scat_wait(0)

    plsc.subcore_barrier()

    pltpu.sync_copy(acc_sh.at[pl.ds(s * STRIPE, STRIPE)],
                    out_hbm.at[pl.ds(c * NN + s * STRIPE, STRIPE)])


def _spmm(dst, src2, values, xcat):
    f = pl.kernel(
        _spmm_body,
        out_type=jax.ShapeDtypeStruct((2 * NN, DH), jnp.float32),
        mesh=_MESH,
        compiler_params=pltpu.CompilerParams(use_tc_tiling_on_sc=False),
        scratch_types=(
            [pltpu.VMEM_SHARED((NN, DH), jnp.float32)]
            + [pltpu.VMEM((CHUNK,), jnp.int32)] * 6
            + [pltpu.VMEM((CHUNK,), jnp.float32)] * 3
            + [pltpu.VMEM((CHUNK, DH), jnp.float32)] * 3
            + [pltpu.SemaphoreType.DMA] * 9
        ),
    )
    return f(dst, src2, values, xcat)


GB = 128


def _head_body(ue0, ie0, u1, u2, i1, i2, users, items, negs,
               o_u1s, o_u2s, o_i1s, o_i2s, o_ue, o_ie, o_ne,
               uidx, iidx, nidx, buf):
    c = lax.axis_index("c")
    s = lax.axis_index("s")
    w = s * 2 + c

    @pl.when(w < B // GB)
    def _():
        b0 = w * GB
        pltpu.sync_copy(users.at[pl.ds(b0, GB)], uidx)
        pltpu.sync_copy(items.at[pl.ds(b0, GB)], iidx)
        pltpu.sync_copy(negs.at[pl.ds(b0, GB)], nidx)
        for tbl, idx, out in ((u1, uidx, o_u1s), (u2, uidx, o_u2s),
                              (i1, iidx, o_i1s), (i2, iidx, o_i2s),
                              (ue0, uidx, o_ue), (ie0, iidx, o_ie),
                              (ie0, nidx, o_ne)):
            pltpu.sync_copy(tbl.at[idx], buf)
            pltpu.sync_copy(buf, out.at[pl.ds(b0, GB)])


def _head(ue0, ie0, u1, u2, i1, i2, users, items, negs):
    rows = jax.ShapeDtypeStruct((B, D), jnp.float32)
    f = pl.kernel(
        _head_body,
        out_type=(rows,) * 7,
        mesh=_MESH,
        compiler_params=pltpu.CompilerParams(use_tc_tiling_on_sc=False),
        scratch_types=[
            pltpu.VMEM((GB,), jnp.int32),
            pltpu.VMEM((GB,), jnp.int32),
            pltpu.VMEM((GB,), jnp.int32),
            pltpu.VMEM((GB, D), jnp.float32),
        ],
    )
    return f(ue0, ie0, u1, u2, i1, i2, users, items, negs)


def _avg4_body(al, ah, bl, bh, cl, ch, dl, dh, o):
    lo = (al[...] + bl[...] + cl[...] + dl[...]) * 0.25
    hi = (ah[...] + bh[...] + ch[...] + dh[...]) * 0.25
    o[...] = jnp.concatenate([lo, hi], axis=1)


def _avg4n_body(al, ah, bl, bh, cl, ch, dl, dh, o):
    lo = (al[...] + bl[...] + cl[...] + dl[...]) * 0.25
    hi = (ah[...] + bh[...] + ch[...] + dh[...]) * 0.25
    m = jnp.concatenate([lo, hi], axis=1)
    n = jnp.sqrt(jnp.sum(m * m, axis=1, keepdims=True))
    o[...] = m / jnp.maximum(n, 1e-12)


def _avg4(x0c, x1c, x2c, x3c, norm):
    body = _avg4n_body if norm else _avg4_body
    R = 1000
    lo = pl.BlockSpec((R, DH), lambda i: (i, 0))
    hi = pl.BlockSpec((R, DH), lambda i: (i + NN // R, 0))
    return pl.pallas_call(
        body,
        grid=(NN // R,),
        in_specs=[lo, hi] * 4,
        out_specs=pl.BlockSpec((R, D), lambda i: (i, 0)),
        out_shape=jax.ShapeDtypeStruct((NN, D), jnp.float32),
    )(x0c, x0c, x1c, x1c, x2c, x2c, x3c, x3c)


def _denoise_body(x, nz, w1, b1, w2, b2, o):
    h = lax.dot_general(x[...] + nz[...], w1[...], (((1,), (0,)), ((), ())),
                        preferred_element_type=jnp.float32) + b1[...]
    h = jnp.maximum(h, 0.0)
    o[...] = lax.dot_general(h, w2[...], (((1,), (0,)), ((), ())),
                             preferred_element_type=jnp.float32) + b2[...]


def _denoise(x, nz, w1, b1, w2, b2):
    R = 1000
    return pl.pallas_call(
        _denoise_body,
        grid=(N_USERS // R,),
        in_specs=[pl.BlockSpec((R, D), lambda i: (i, 0)),
                  pl.BlockSpec((R, D), lambda i: (i, 0)),
                  pl.BlockSpec((D, 2 * D), lambda i: (0, 0)),
                  pl.BlockSpec((1, 2 * D), lambda i: (0, 0)),
                  pl.BlockSpec((2 * D, D), lambda i: (0, 0)),
                  pl.BlockSpec((1, D), lambda i: (0, 0))],
        out_specs=pl.BlockSpec((R, D), lambda i: (i, 0)),
        out_shape=jax.ShapeDtypeStruct((N_USERS, D), jnp.float32),
    )(x, nz, w1, b1.reshape(1, -1), w2, b2.reshape(1, -1))


def _sup_body(ue, ie, ne, o):
    o[...] = jnp.sum(ue[...] * (ie[...] - ne[...]), axis=1, keepdims=True)


def _sup(ue, ie, ne):
    return pl.pallas_call(
        _sup_body,
        out_shape=jax.ShapeDtypeStruct((B, 1), jnp.float32),
    )(ue, ie, ne).reshape(B)


def _ssl_body(u1s, u2s, tbl, o):
    pos = jnp.sum(u1s[...] * u2s[...], axis=1, keepdims=True)
    o[...] = lax.dot_general(u1s[...], tbl[...], (((1,), (1,)), ((), ())),
                             preferred_element_type=jnp.float32) - pos


def _ssl(u1s, u2s, tbl):
    TB = 1024
    NP = tbl.shape[0]
    return pl.pallas_call(
        _ssl_body,
        grid=(NP // TB,),
        in_specs=[pl.BlockSpec((B, D), lambda i: (0, 0)),
                  pl.BlockSpec((B, D), lambda i: (0, 0)),
                  pl.BlockSpec((TB, D), lambda i: (i, 0))],
        out_specs=pl.BlockSpec((B, TB), lambda i: (0, i)),
        out_shape=jax.ShapeDtypeStruct((B, NP), jnp.float32),
    )(u1s, u2s, tbl)


def kernel(adj_indices, adj_values, sub1_indices, sub1_values,
           sub2_indices, sub2_values, users, items, neg_items,
           user_emb, item_emb,
           du_W1, du_b1, du_W2, du_b2, di_W1, di_b1, di_W2, di_b2):
    adj_indices = adj_indices.astype(jnp.int32)
    sub1_indices = sub1_indices.astype(jnp.int32)
    sub2_indices = sub2_indices.astype(jnp.int32)
    users = users.astype(jnp.int32)
    items = items.astype(jnp.int32)
    negs = neg_items.astype(jnp.int32)

    x0 = jnp.concatenate([user_emb, item_emb], axis=0)
    x0c = jnp.concatenate([x0[:, :DH], x0[:, DH:]], axis=0)

    def chain(idx, val, norm):
        dst = idx[0]
        src2 = jnp.stack([idx[1], idx[1] + NN])
        x1 = _spmm(dst, src2, val, x0c)
        x2 = _spmm(dst, src2, val, x1)
        x3 = _spmm(dst, src2, val, x2)
        return _avg4(x0c, x1, x2, x3, norm)

    avg_adj = chain(adj_indices, adj_values, False)
    avg1 = chain(sub1_indices, sub1_values, True)
    avg2 = chain(sub2_indices, sub2_values, True)

    nkey = jax.random.key(42)
    nz_u = jax.random.normal(jax.random.fold_in(nkey, 0),
                             (N_USERS, D), dtype=jnp.float32) * 0.1
    nz_i = jax.random.normal(jax.random.fold_in(nkey, 1),
                             (N_ITEMS, D), dtype=jnp.float32) * 0.1

    ue0 = _denoise(avg_adj[:N_USERS], nz_u, du_W1, du_b1, du_W2, du_b2)
    ie0 = _denoise(avg_adj[N_USERS:], nz_i, di_W1, di_b1, di_W2, di_b2)

    u1, i1 = avg1[:N_USERS], avg1[N_USERS:]
    u2, i2 = avg2[:N_USERS], avg2[N_USERS:]

    u1s, u2s, i1s, i2s, u_e, i_e, n_e = _head(
        ue0, ie0, u1, u2, i1, i2, users, items, negs)

    sup = _sup(u_e, i_e, n_e)
    u2p = jnp.pad(u2, ((0, 600), (0, 0)))
    i2p = jnp.pad(i2, ((0, 600), (0, 0)))
    ssl_u = _ssl(u1s, u2s, u2p)[:, :N_USERS]
    ssl_i = _ssl(i1s, i2s, i2p)[:, :N_ITEMS]
    return sup, ssl_u, ssl_i


# speedup vs baseline: 7.8379x; 1.3295x over previous
import jax
import jax.numpy as jnp
from jax import lax
from jax.experimental import pallas as pl
from jax.experimental.pallas import tpu as pltpu
from jax.experimental.pallas import tpu_sc as plsc

N_USERS = 25000
N_ITEMS = 25000
NN = 50000
D = 64
E = 800000
B = 1024

DH = D // 2
IW = 128
KJ = 2
CHUNK = KJ * IW
NCHUNKS = E // CHUNK
FULL = NCHUNKS // 16
EXTRA = NCHUNKS - FULL * 16
STRIPE = NN // 16
NBUF = 3

_MESH = plsc.VectorSubcoreMesh(core_axis_name="c", subcore_axis_name="s")


def _spmm_body(dst_hbm, src2_hbm, val_hbm, x_hbm, out_hbm, acc_sh,
               src0, src1, src2, dl0, dl1, dl2, val0, val1, val2,
               rows0, rows1, rows2,
               sin0, sin1, sin2, srow0, srow1, srow2, ssc0, ssc1, ssc2):
    c = lax.axis_index("c")
    s = lax.axis_index("s")

    srcb = (src0, src1, src2)
    dlb = (dl0, dl1, dl2)
    valb = (val0, val1, val2)
    rowsb = (rows0, rows1, rows2)
    sin = (sin0, sin1, sin2)
    srow = (srow0, srow1, srow2)
    ssc = (ssc0, ssc1, ssc2)

    @pl.loop(0, CHUNK)
    def _(e):
        for k in range(DH // 16):
            rows0[e, pl.ds(k * 16, 16)] = jnp.zeros((16,), jnp.float32)

    z0 = s * STRIPE
    nz_full = STRIPE // CHUNK
    nz_rem = STRIPE - nz_full * CHUNK

    for i in range(nz_full):
        pltpu.async_copy(rows0, acc_sh.at[pl.ds(z0 + i * CHUNK, CHUNK)], sin0)
    if nz_rem:
        pltpu.async_copy(rows0.at[pl.ds(0, nz_rem)],
                         acc_sh.at[pl.ds(z0 + nz_full * CHUNK, nz_rem)], sin0)
    for i in range(nz_full):
        pltpu.make_async_copy(
            rows0, acc_sh.at[pl.ds(z0 + i * CHUNK, CHUNK)], sin0).wait()
    if nz_rem:
        pltpu.make_async_copy(
            rows0.at[pl.ds(0, nz_rem)],
            acc_sh.at[pl.ds(z0 + nz_full * CHUNK, nz_rem)], sin0).wait()

    plsc.subcore_barrier()

    def fetch(m, b):
        r0 = m * KJ
        pltpu.async_copy(src2_hbm.at[c, pl.ds(r0, KJ)], srcb[b], sin[b])
        pltpu.async_copy(dst_hbm.at[pl.ds(r0, KJ)], dlb[b], sin[b])
        pltpu.async_copy(val_hbm.at[pl.ds(r0, KJ)], valb[b], sin[b])

    def fetch_wait(b):
        pltpu.make_async_copy(src2_hbm.at[0, pl.ds(0, KJ)], srcb[b], sin[b]).wait()
        pltpu.make_async_copy(dst_hbm.at[pl.ds(0, KJ)], dlb[b], sin[b]).wait()
        pltpu.make_async_copy(val_hbm.at[pl.ds(0, KJ)], valb[b], sin[b]).wait()

    def gather_start(b):
        for j in range(KJ):
            pltpu.async_copy(x_hbm.at[srcb[b].at[j]],
                             rowsb[b].at[pl.ds(j * IW, IW)], srow[b])

    def gather_wait(b):
        for j in range(KJ):
            pltpu.make_async_copy(x_hbm.at[srcb[b].at[j]],
                                  rowsb[b].at[pl.ds(j * IW, IW)], srow[b]).wait()

    def scat_start(b):
        for j in range(KJ):
            pltpu.async_copy(rowsb[b].at[pl.ds(j * IW, IW)],
                             acc_sh.at[dlb[b].at[j]], ssc[b], add=True)

    def scat_wait(b):
        for j in range(KJ):
            pltpu.make_async_copy(rowsb[b].at[pl.ds(j * IW, IW)],
                                  acc_sh.at[dlb[b].at[j]], ssc[b]).wait()

    def compute(b):
        for j in range(KJ):
            @pl.loop(0, IW, step=16)
            def _(i):
                v16 = valb[b][j, pl.ds(i, 16)]
                for u in range(16):
                    v = v16[u]
                    for k in range(DH // 16):
                        sl = pl.ds(k * 16, 16)
                        rowsb[b][j * IW + i + u, sl] = (
                            rowsb[b][j * IW + i + u, sl] * v)

    def mid(g):
        return g * 16 + s

    fetch(mid(0), 0)
    fetch_wait(0)
    gather_start(0)
    fetch(mid(1), 1)

    @pl.loop(0, FULL, step=NBUF)
    def _(g0):
        for b in range(NBUF):
            g = g0 + b
            bp1 = (b + 1) % NBUF
            bp2 = (b + 2) % NBUF

            @pl.when(g + 1 < FULL)
            def _():
                fetch_wait(bp1)

                @pl.when(g >= 2)
                def _():
                    scat_wait(bp1)

                gather_start(bp1)

            gather_wait(b)

            @pl.when(g + 2 < FULL)
            def _():
                fetch(mid(g + 2), bp2)

            compute(b)
            scat_start(b)

    scat_wait(0)
    scat_wait(1)
    scat_wait(2)

    @pl.when(s < EXTRA)
    def _():
        fetch(mid(FULL), 0)
        fetch_wait(0)
        gather_start(0)
        gather_wait(0)
        compute(0)
        scat_start(0)
        scat_wait(0)

    plsc.subcore_barrier()

    pltpu.sync_copy(acc_sh.at[pl.ds(s * STRIPE, STRIPE)],
                    out_hbm.at[pl.ds(c * NN + s * STRIPE, STRIPE)])


def _spmm(dst, src2, values, xcat):
    f = pl.kernel(
        _spmm_body,
        out_type=jax.ShapeDtypeStruct((2 * NN, DH), jnp.float32),
        mesh=_MESH,
        compiler_params=pltpu.CompilerParams(use_tc_tiling_on_sc=False),
        scratch_types=(
            [pltpu.VMEM_SHARED((NN, DH), jnp.float32)]
            + [pltpu.VMEM((KJ, IW), jnp.int32)] * 6
            + [pltpu.VMEM((KJ, IW), jnp.float32)] * 3
            + [pltpu.VMEM((CHUNK, DH), jnp.float32)] * 3
            + [pltpu.SemaphoreType.DMA] * 9
        ),
    )
    return f(dst, src2, values, xcat)


GB = 128


def _head_body(ue0, ie0, u1, u2, i1, i2, users, items, negs,
               o_u1s, o_u2s, o_i1s, o_i2s, o_ue, o_ie, o_ne,
               uidx, iidx, nidx, buf):
    c = lax.axis_index("c")
    s = lax.axis_index("s")
    w = s * 2 + c

    @pl.when(w < B // GB)
    def _():
        b0 = w * GB
        pltpu.sync_copy(users.at[pl.ds(b0, GB)], uidx)
        pltpu.sync_copy(items.at[pl.ds(b0, GB)], iidx)
        pltpu.sync_copy(negs.at[pl.ds(b0, GB)], nidx)
        for tbl, idx, out in ((u1, uidx, o_u1s), (u2, uidx, o_u2s),
                              (i1, iidx, o_i1s), (i2, iidx, o_i2s),
                              (ue0, uidx, o_ue), (ie0, iidx, o_ie),
                              (ie0, nidx, o_ne)):
            pltpu.sync_copy(tbl.at[idx], buf)
            pltpu.sync_copy(buf, out.at[pl.ds(b0, GB)])


def _head(ue0, ie0, u1, u2, i1, i2, users, items, negs):
    rows = jax.ShapeDtypeStruct((B, D), jnp.float32)
    f = pl.kernel(
        _head_body,
        out_type=(rows,) * 7,
        mesh=_MESH,
        compiler_params=pltpu.CompilerParams(use_tc_tiling_on_sc=False),
        scratch_types=[
            pltpu.VMEM((GB,), jnp.int32),
            pltpu.VMEM((GB,), jnp.int32),
            pltpu.VMEM((GB,), jnp.int32),
            pltpu.VMEM((GB, D), jnp.float32),
        ],
    )
    return f(ue0, ie0, u1, u2, i1, i2, users, items, negs)


def _avg4_body(al, ah, bl, bh, cl, ch, dl, dh, o):
    lo = (al[...] + bl[...] + cl[...] + dl[...]) * 0.25
    hi = (ah[...] + bh[...] + ch[...] + dh[...]) * 0.25
    o[...] = jnp.concatenate([lo, hi], axis=1)


def _avg4n_body(al, ah, bl, bh, cl, ch, dl, dh, o):
    lo = (al[...] + bl[...] + cl[...] + dl[...]) * 0.25
    hi = (ah[...] + bh[...] + ch[...] + dh[...]) * 0.25
    m = jnp.concatenate([lo, hi], axis=1)
    n = jnp.sqrt(jnp.sum(m * m, axis=1, keepdims=True))
    o[...] = m / jnp.maximum(n, 1e-12)


def _avg4(x0c, x1c, x2c, x3c, norm):
    body = _avg4n_body if norm else _avg4_body
    R = 1000
    lo = pl.BlockSpec((R, DH), lambda i: (i, 0))
    hi = pl.BlockSpec((R, DH), lambda i: (i + NN // R, 0))
    return pl.pallas_call(
        body,
        grid=(NN // R,),
        in_specs=[lo, hi] * 4,
        out_specs=pl.BlockSpec((R, D), lambda i: (i, 0)),
        out_shape=jax.ShapeDtypeStruct((NN, D), jnp.float32),
    )(x0c, x0c, x1c, x1c, x2c, x2c, x3c, x3c)


def _denoise_body(x, nz, w1, b1, w2, b2, o):
    h = lax.dot_general(x[...] + nz[...], w1[...], (((1,), (0,)), ((), ())),
                        preferred_element_type=jnp.float32) + b1[...]
    h = jnp.maximum(h, 0.0)
    o[...] = lax.dot_general(h, w2[...], (((1,), (0,)), ((), ())),
                             preferred_element_type=jnp.float32) + b2[...]


def _denoise(x, nz, w1, b1, w2, b2):
    R = 1000
    return pl.pallas_call(
        _denoise_body,
        grid=(N_USERS // R,),
        in_specs=[pl.BlockSpec((R, D), lambda i: (i, 0)),
                  pl.BlockSpec((R, D), lambda i: (i, 0)),
                  pl.BlockSpec((D, 2 * D), lambda i: (0, 0)),
                  pl.BlockSpec((1, 2 * D), lambda i: (0, 0)),
                  pl.BlockSpec((2 * D, D), lambda i: (0, 0)),
                  pl.BlockSpec((1, D), lambda i: (0, 0))],
        out_specs=pl.BlockSpec((R, D), lambda i: (i, 0)),
        out_shape=jax.ShapeDtypeStruct((N_USERS, D), jnp.float32),
    )(x, nz, w1, b1.reshape(1, -1), w2, b2.reshape(1, -1))


def _sup_body(ue, ie, ne, o):
    o[...] = jnp.sum(ue[...] * (ie[...] - ne[...]), axis=1, keepdims=True)


def _sup(ue, ie, ne):
    return pl.pallas_call(
        _sup_body,
        out_shape=jax.ShapeDtypeStruct((B, 1), jnp.float32),
    )(ue, ie, ne).reshape(B)


def _ssl_body(u1s, u2s, tbl, o):
    pos = jnp.sum(u1s[...] * u2s[...], axis=1, keepdims=True)
    o[...] = lax.dot_general(u1s[...], tbl[...], (((1,), (1,)), ((), ())),
                             preferred_element_type=jnp.float32) - pos


def _ssl(u1s, u2s, tbl):
    TB = 1024
    NP = tbl.shape[0]
    return pl.pallas_call(
        _ssl_body,
        grid=(NP // TB,),
        in_specs=[pl.BlockSpec((B, D), lambda i: (0, 0)),
                  pl.BlockSpec((B, D), lambda i: (0, 0)),
                  pl.BlockSpec((TB, D), lambda i: (i, 0))],
        out_specs=pl.BlockSpec((B, TB), lambda i: (0, i)),
        out_shape=jax.ShapeDtypeStruct((B, NP), jnp.float32),
    )(u1s, u2s, tbl)


def kernel(adj_indices, adj_values, sub1_indices, sub1_values,
           sub2_indices, sub2_values, users, items, neg_items,
           user_emb, item_emb,
           du_W1, du_b1, du_W2, du_b2, di_W1, di_b1, di_W2, di_b2):
    adj_indices = adj_indices.astype(jnp.int32)
    sub1_indices = sub1_indices.astype(jnp.int32)
    sub2_indices = sub2_indices.astype(jnp.int32)
    users = users.astype(jnp.int32)
    items = items.astype(jnp.int32)
    negs = neg_items.astype(jnp.int32)

    x0 = jnp.concatenate([user_emb, item_emb], axis=0)
    x0c = jnp.concatenate([x0[:, :DH], x0[:, DH:]], axis=0)

    def chain(idx, val, norm):
        dst = idx[0].reshape(E // IW, IW)
        src2 = jnp.stack([idx[1], idx[1] + NN]).reshape(2, E // IW, IW)
        val = val.reshape(E // IW, IW)
        x1 = _spmm(dst, src2, val, x0c)
        x2 = _spmm(dst, src2, val, x1)
        x3 = _spmm(dst, src2, val, x2)
        return _avg4(x0c, x1, x2, x3, norm)

    avg_adj = chain(adj_indices, adj_values, False)
    avg1 = chain(sub1_indices, sub1_values, True)
    avg2 = chain(sub2_indices, sub2_values, True)

    nkey = jax.random.key(42)
    nz_u = jax.random.normal(jax.random.fold_in(nkey, 0),
                             (N_USERS, D), dtype=jnp.float32) * 0.1
    nz_i = jax.random.normal(jax.random.fold_in(nkey, 1),
                             (N_ITEMS, D), dtype=jnp.float32) * 0.1

    ue0 = _denoise(avg_adj[:N_USERS], nz_u, du_W1, du_b1, du_W2, du_b2)
    ie0 = _denoise(avg_adj[N_USERS:], nz_i, di_W1, di_b1, di_W2, di_b2)

    u1, i1 = avg1[:N_USERS], avg1[N_USERS:]
    u2, i2 = avg2[:N_USERS], avg2[N_USERS:]

    u1s, u2s, i1s, i2s, u_e, i_e, n_e = _head(
        ue0, ie0, u1, u2, i1, i2, users, items, negs)

    sup = _sup(u_e, i_e, n_e)
    u2p = jnp.pad(u2, ((0, 600), (0, 0)))
    i2p = jnp.pad(i2, ((0, 600), (0, 0)))
    ssl_u = _ssl(u1s, u2s, u2p)[:, :N_USERS]
    ssl_i = _ssl(i1s, i2s, i2p)[:, :N_ITEMS]
    return sup, ssl_u, ssl_i
